# initial kernel scaffold (unmeasured)
import jax
import jax.numpy as jnp
from jax import lax
from jax.experimental import pallas as pl
from jax.experimental.pallas import tpu as pltpu


def kernel(
    x,
):
    def body(*refs):
        pass

    out_shape = jax.ShapeDtypeStruct(..., jnp.float32)
    return pl.pallas_call(body, out_shape=out_shape)(...)


# baseline (device time: 15261 ns/iter reference)
import jax
import jax.numpy as jnp
from jax import lax
from jax.experimental import pallas as pl
from jax.experimental.pallas import tpu as pltpu

N_DEV = 8


def kernel(x):
    m_per, n = x.shape

    def body(x_ref, out_ref, comm_ref, send_sems, recv_sems):
        my = lax.axis_index("i")

        comm_ref[pl.ds(my, 1), :] = jnp.max(x_ref[:, :], axis=0, keepdims=True)

        rdmas = []
        for off in range(1, N_DEV):
            peer = lax.rem(my + off, N_DEV)
            rdma = pltpu.make_async_remote_copy(
                src_ref=comm_ref.at[pl.ds(my, 1)],
                dst_ref=comm_ref.at[pl.ds(my, 1)],
                send_sem=send_sems.at[off - 1],
                recv_sem=recv_sems.at[off - 1],
                device_id=(peer,),
                device_id_type=pl.DeviceIdType.MESH,
            )
            rdma.start()
            rdmas.append(rdma)
        for rdma in rdmas:
            rdma.wait()

        out_ref[:, :] = jnp.max(comm_ref[:, :], axis=0, keepdims=True)

    return pl.pallas_call(
        body,
        out_shape=jax.ShapeDtypeStruct((1, n), x.dtype),
        in_specs=[pl.BlockSpec(memory_space=pltpu.VMEM)],
        out_specs=pl.BlockSpec(memory_space=pltpu.VMEM),
        scratch_shapes=[
            pltpu.VMEM((N_DEV, n), x.dtype),
            pltpu.SemaphoreType.DMA((N_DEV - 1,)),
            pltpu.SemaphoreType.DMA((N_DEV - 1,)),
        ],
    )(x)


# device time: 15141 ns/iter; 1.0079x vs baseline; 1.0079x over previous
import jax
import jax.numpy as jnp
from jax import lax
from jax.experimental import pallas as pl
from jax.experimental.pallas import tpu as pltpu

N_DEV = 8
GRID = 8


def kernel(x):
    m_per, n = x.shape
    bm = m_per // GRID

    def body(x_ref, out_ref, acc_ref, comm_ref, send_sems, recv_sems):
        g = pl.program_id(0)
        blk = jnp.max(x_ref[:, :], axis=0, keepdims=True)

        @pl.when(g == 0)
        def _():
            acc_ref[:, :] = blk

        @pl.when(g > 0)
        def _():
            acc_ref[:, :] = jnp.maximum(acc_ref[:, :], blk)

        @pl.when(g == GRID - 1)
        def _():
            my = lax.axis_index("i")
            comm_ref[pl.ds(my, 1), :] = acc_ref[:, :]

            rdmas = []
            for off in range(1, N_DEV):
                peer = lax.rem(my + off, N_DEV)
                rdma = pltpu.make_async_remote_copy(
                    src_ref=comm_ref.at[pl.ds(my, 1)],
                    dst_ref=comm_ref.at[pl.ds(my, 1)],
                    send_sem=send_sems.at[off - 1],
                    recv_sem=recv_sems.at[off - 1],
                    device_id=(peer,),
                    device_id_type=pl.DeviceIdType.MESH,
                )
                rdma.start()
                rdmas.append(rdma)
            for rdma in rdmas:
                rdma.wait()

            out_ref[:, :] = jnp.max(comm_ref[:, :], axis=0, keepdims=True)

    return pl.pallas_call(
        body,
        grid=(GRID,),
        out_shape=jax.ShapeDtypeStruct((1, n), x.dtype),
        in_specs=[pl.BlockSpec((bm, n), lambda g: (g, 0))],
        out_specs=pl.BlockSpec((1, n), lambda g: (0, 0)),
        scratch_shapes=[
            pltpu.VMEM((1, n), x.dtype),
            pltpu.VMEM((N_DEV, n), x.dtype),
            pltpu.SemaphoreType.DMA((N_DEV - 1,)),
            pltpu.SemaphoreType.DMA((N_DEV - 1,)),
        ],
    )(x)
